# baseline (speedup 1.0000x reference)
import functools

import jax
import jax.numpy as jnp
from jax import lax
from jax.experimental import pallas as pl
from jax.experimental.pallas import tpu as pltpu
from jax.experimental.pallas import tpu_sc as plsc

DIM = 128
NC, NS = 2, 16
NW = NC * NS
CHUNK = 64
B_TOTAL = 4096 * 50
B_PER_W = B_TOTAL // NW
NSTEP = B_PER_W // CHUNK
NBUF = 4

_mesh = plsc.VectorSubcoreMesh(core_axis_name="c", subcore_axis_name="s")


@functools.partial(
    pl.kernel,
    out_type=jax.ShapeDtypeStruct((B_TOTAL, DIM), jnp.float32),
    mesh=_mesh,
    scratch_types=[
        pltpu.VMEM((NSTEP, CHUNK), jnp.int32),
        pltpu.VMEM((NBUF, CHUNK, DIM), jnp.float32),
        pltpu.VMEM_SHARED((8192, DIM), jnp.float32),
        [pltpu.SemaphoreType.DMA] * NBUF,
        [pltpu.SemaphoreType.DMA] * NBUF,
        pltpu.SemaphoreType.DMA,
    ],
)
def _gather(enc_hbm, pos_hbm, out_hbm, idx_v, rows_v, enc_sh, gsem, ssem, stsem):
    wid = lax.axis_index("s") * NC + lax.axis_index("c")
    base = wid * B_PER_W

    sid = lax.axis_index("s")
    stage = 8192 // NS
    staging = pltpu.async_copy(enc_hbm.at[pl.ds(sid * stage, stage)],
                               enc_sh.at[pl.ds(sid * stage, stage)], stsem)
    pltpu.sync_copy(pos_hbm.at[wid], idx_v)
    staging.wait()
    plsc.subcore_barrier()

    def sg(j, b):
        pltpu.async_copy(enc_sh.at[idx_v.at[j]], rows_v.at[b], gsem[b])

    def wg(j, b):
        pltpu.make_async_copy(enc_sh.at[idx_v.at[j]], rows_v.at[b], gsem[b]).wait()

    def out_slice(j):
        return out_hbm.at[pl.ds(base + j * CHUNK, CHUNK)]

    def ss(j, b):
        pltpu.async_copy(rows_v.at[b], out_slice(j), ssem[b])

    def ws(j, b):
        pltpu.make_async_copy(rows_v.at[b], out_slice(j), ssem[b]).wait()

    sg(0, 0)
    sg(1, 1)

    for j in range(NBUF):
        bb = (j + 2) % NBUF
        if j + 2 >= NBUF:
            ws(j + 2 - NBUF, bb)
        sg(j + 2, bb)
        wg(j, j % NBUF)
        ss(j, j % NBUF)

    NSTEADY = ((NSTEP - NBUF - 2) // NBUF) * NBUF

    @pl.loop(NBUF, NBUF + NSTEADY, step=NBUF)
    def _(t):
        for b in range(NBUF):
            j = t + b
            bb = (b + 2) % NBUF
            ws(j + 2 - NBUF, bb)
            sg(j + 2, bb)
            wg(j, b)
            ss(j, b)

    for j in range(NBUF + NSTEADY, NSTEP):
        b = j % NBUF
        bb = (b + 2) % NBUF
        if j + 2 < NSTEP:
            ws(j + 2 - NBUF, bb)
            sg(j + 2, bb)
        wg(j, b)
        ss(j, b)

    for j in range(NSTEP - NBUF, NSTEP):
        ws(j, j % NBUF)


def kernel(positions, encodings):
    pos = positions.reshape(NW, NSTEP, CHUNK)
    out = _gather(encodings, pos)
    return out.reshape(4096, 1, 50, DIM)


# speedup vs baseline: 1.0220x; 1.0220x over previous
import functools

import jax
import jax.numpy as jnp
from jax import lax
from jax.experimental import pallas as pl
from jax.experimental.pallas import tpu as pltpu
from jax.experimental.pallas import tpu_sc as plsc

DIM = 128
NC, NS = 2, 16
NW = NC * NS
CHUNK = 80
B_TOTAL = 4096 * 50
B_PER_W = B_TOTAL // NW
NSTEP = B_PER_W // CHUNK
NBUF = 5

_mesh = plsc.VectorSubcoreMesh(core_axis_name="c", subcore_axis_name="s")


@functools.partial(
    pl.kernel,
    out_type=jax.ShapeDtypeStruct((B_TOTAL, DIM), jnp.float32),
    mesh=_mesh,
    scratch_types=[
        pltpu.VMEM((NSTEP, CHUNK), jnp.int32),
        pltpu.VMEM((NBUF, CHUNK, DIM), jnp.float32),
        pltpu.VMEM_SHARED((8192, DIM), jnp.float32),
        [pltpu.SemaphoreType.DMA] * NBUF,
        [pltpu.SemaphoreType.DMA] * NBUF,
        pltpu.SemaphoreType.DMA,
    ],
)
def _gather(enc_hbm, pos_hbm, out_hbm, idx_v, rows_v, enc_sh, gsem, ssem, stsem):
    wid = lax.axis_index("s") * NC + lax.axis_index("c")
    base = wid * B_PER_W

    sid = lax.axis_index("s")
    stage = 8192 // NS
    staging = pltpu.async_copy(enc_hbm.at[pl.ds(sid * stage, stage)],
                               enc_sh.at[pl.ds(sid * stage, stage)], stsem)
    pltpu.sync_copy(pos_hbm.at[wid], idx_v)
    staging.wait()
    plsc.subcore_barrier()

    def sg(j, b):
        pltpu.async_copy(enc_sh.at[idx_v.at[j]], rows_v.at[b], gsem[b])

    def wg(j, b):
        pltpu.make_async_copy(enc_sh.at[idx_v.at[j]], rows_v.at[b], gsem[b]).wait()

    def out_slice(j):
        return out_hbm.at[pl.ds(base + j * CHUNK, CHUNK)]

    def ss(j, b):
        pltpu.async_copy(rows_v.at[b], out_slice(j), ssem[b])

    def ws(j, b):
        pltpu.make_async_copy(rows_v.at[b], out_slice(j), ssem[b]).wait()

    sg(0, 0)
    sg(1, 1)

    for j in range(NBUF):
        bb = (j + 2) % NBUF
        if j + 2 >= NBUF:
            ws(j + 2 - NBUF, bb)
        sg(j + 2, bb)
        wg(j, j % NBUF)
        ss(j, j % NBUF)

    NSTEADY = ((NSTEP - NBUF - 2) // NBUF) * NBUF

    @pl.loop(NBUF, NBUF + NSTEADY, step=NBUF)
    def _(t):
        for b in range(NBUF):
            j = t + b
            bb = (b + 2) % NBUF
            ws(j + 2 - NBUF, bb)
            sg(j + 2, bb)
            wg(j, b)
            ss(j, b)

    for j in range(NBUF + NSTEADY, NSTEP):
        b = j % NBUF
        bb = (b + 2) % NBUF
        if j + 2 < NSTEP:
            ws(j + 2 - NBUF, bb)
            sg(j + 2, bb)
        wg(j, b)
        ss(j, b)

    for j in range(NSTEP - NBUF, NSTEP):
        ws(j, j % NBUF)


def kernel(positions, encodings):
    pos = positions.reshape(NW, NSTEP, CHUNK)
    out = _gather(encodings, pos)
    return out.reshape(4096, 1, 50, DIM)
